# baseline (speedup 1.0000x reference)
import functools

import jax
import jax.numpy as jnp
from jax import lax
from jax.experimental import pallas as pl
from jax.experimental.pallas import tpu as pltpu
from jax.experimental.pallas import tpu_sc as plsc

N_BUCKETS = 1000000
BIGRAM_DIM = 64
D_MODEL = 1024
B, S = 4, 4096
N = B * S

NC, NS, L = 2, 16, 16
NW = NC * NS
CHUNK = N // NW
NVEC = CHUNK // L
GBATCH = 128
NBATCH = CHUNK // GBATCH
PAIRW = 2 * BIGRAM_DIM

_sc_mesh = plsc.VectorSubcoreMesh(core_axis_name="c", subcore_axis_name="s")


def _tc_pack_body(tab_t_ref, out_ref):
    blk = tab_t_ref[...].T
    blk3 = blk.reshape(blk.shape[0] // 2, 2, BIGRAM_DIM)
    out_ref[...] = jnp.concatenate([blk3[:, 0, :], blk3[:, 1, :]], axis=-1)


_TBLK = 8192
_tc_pack = pl.pallas_call(
    _tc_pack_body,
    grid=(pl.cdiv(N_BUCKETS, _TBLK),),
    in_specs=[pl.BlockSpec((BIGRAM_DIM, _TBLK), lambda i: (0, i))],
    out_specs=pl.BlockSpec((_TBLK // 2, PAIRW), lambda i: (i, 0)),
    out_shape=jax.ShapeDtypeStruct((N_BUCKETS // 2, PAIRW), jnp.float32),
)


@functools.partial(
    pl.kernel,
    out_type=jax.ShapeDtypeStruct((N, BIGRAM_DIM), jnp.float32),
    mesh=_sc_mesh,
    scratch_types=[
        pltpu.VMEM((CHUNK + L,), jnp.int32),
        pltpu.VMEM((CHUNK + L,), jnp.int32),
        pltpu.VMEM((GBATCH, PAIRW), jnp.float32),
        pltpu.VMEM((CHUNK, BIGRAM_DIM), jnp.float32),
        pltpu.SemaphoreType.DMA,
    ],
)
def _sc_hash_gather(ids_hbm, table_hbm, out_hbm, ids_ext, hv, pairs_v, rows_v, sem_g):
    wid = lax.axis_index("s") * NC + lax.axis_index("c")
    base = wid * CHUNK

    pltpu.sync_copy(ids_hbm.at[pl.ds(base, CHUNK)], ids_ext.at[pl.ds(L, CHUNK)])

    @pl.when(wid != 0)
    def _():
        pltpu.sync_copy(ids_hbm.at[pl.ds(base - L, L)], ids_ext.at[pl.ds(0, L)])

    lane = lax.iota(jnp.int32, L)
    rs = 1 - jnp.clip(base % S, 0, 1)
    keep0 = 1 - rs * jnp.clip(1 - lane, 0, 1)
    for i in range(NVEC):
        cur = ids_ext[pl.ds(L + i * L, L)]
        prv = ids_ext[pl.ds(L - 1 + i * L, L)]
        if i == 0:
            prv = prv * keep0
        h = (prv * 1009 + cur) % N_BUCKETS
        hv[pl.ds(i * L, L)] = h

    def enqueue(t, _):
        h = hv[pl.ds(t, L)][0]
        pltpu.make_async_copy(
            table_hbm.at[pl.ds(h >> 1, 1), :],
            pairs_v.at[pl.ds(t % GBATCH, 1), :],
            sem_g,
        ).start()
        return 0

    def drain(t, _):
        pltpu.make_async_copy(
            table_hbm.at[pl.ds(0, 1), :],
            pairs_v.at[pl.ds(0, 1), :],
            sem_g,
        ).wait()
        return 0

    def select(t, _):
        h = hv[pl.ds(t, L)][0]
        off = (h & 1) * BIGRAM_DIM
        for j in range(BIGRAM_DIM // L):
            rows_v[t, pl.ds(j * L, L)] = pairs_v[t % GBATCH, pl.ds(off + j * L, L)]
        return 0

    for b in range(NBATCH):
        lax.fori_loop(b * GBATCH, (b + 1) * GBATCH, enqueue, 0, unroll=8)
        lax.fori_loop(0, GBATCH, drain, 0, unroll=8)
        lax.fori_loop(b * GBATCH, (b + 1) * GBATCH, select, 0, unroll=4)

    pltpu.sync_copy(rows_v, out_hbm.at[pl.ds(base, CHUNK)])


def _tc_matmul_body(emb_ref, proj_ref, out_ref):
    out_ref[...] = lax.dot_general(
        emb_ref[...],
        proj_ref[...],
        (((1,), (1,)), ((), ())),
        preferred_element_type=jnp.float32,
    )


_ROWS_BLK = 2048
_tc_matmul = pl.pallas_call(
    _tc_matmul_body,
    grid=(N // _ROWS_BLK,),
    in_specs=[
        pl.BlockSpec((_ROWS_BLK, BIGRAM_DIM), lambda i: (i, 0)),
        pl.BlockSpec((D_MODEL, BIGRAM_DIM), lambda i: (0, 0)),
    ],
    out_specs=pl.BlockSpec((_ROWS_BLK, D_MODEL), lambda i: (i, 0)),
    out_shape=jax.ShapeDtypeStruct((N, D_MODEL), jnp.float32),
)


@jax.jit
def kernel(ids, embed_weight, proj_weight):
    ids_flat = ids.reshape(N).astype(jnp.int32)
    table_pairs = _tc_pack(embed_weight.T)
    emb = _sc_hash_gather(ids_flat, table_pairs)
    out = _tc_matmul(emb, proj_weight)
    return out.reshape(B, S, D_MODEL)


# speedup vs baseline: 1.4225x; 1.4225x over previous
import functools

import jax
import jax.numpy as jnp
from jax import lax
from jax.experimental import pallas as pl
from jax.experimental.pallas import tpu as pltpu
from jax.experimental.pallas import tpu_sc as plsc

N_BUCKETS = 1000000
BIGRAM_DIM = 64
D_MODEL = 1024
B, S = 4, 4096
N = B * S

NC, NS, L = 2, 16, 16
NW = NC * NS
CHUNK = N // NW
NVEC = CHUNK // L
GBATCH = 128
NBATCH = CHUNK // GBATCH
PAIRW = 2 * BIGRAM_DIM

_sc_mesh = plsc.VectorSubcoreMesh(core_axis_name="c", subcore_axis_name="s")


def _tc_pack_body(tab_t_ref, out_ref):
    out_ref[:, 0:BIGRAM_DIM] = tab_t_ref[:, 0:_THALF].T
    out_ref[:, BIGRAM_DIM:PAIRW] = tab_t_ref[:, _THALF:_TBLK].T


_TBLK = 8192
_THALF = _TBLK // 2
_TGRID = pl.cdiv(N_BUCKETS, _TBLK)
_PROWS = _TGRID * _THALF
_tc_pack = pl.pallas_call(
    _tc_pack_body,
    grid=(_TGRID,),
    in_specs=[pl.BlockSpec((BIGRAM_DIM, _TBLK), lambda i: (0, i))],
    out_specs=pl.BlockSpec((_THALF, PAIRW), lambda i: (i, 0)),
    out_shape=jax.ShapeDtypeStruct((_PROWS, PAIRW), jnp.float32),
)


@functools.partial(
    pl.kernel,
    out_type=jax.ShapeDtypeStruct((N, BIGRAM_DIM), jnp.float32),
    mesh=_sc_mesh,
    scratch_types=[
        pltpu.VMEM((CHUNK + L,), jnp.int32),
        pltpu.VMEM((CHUNK + L,), jnp.int32),
        pltpu.VMEM((GBATCH, PAIRW), jnp.float32),
        pltpu.VMEM((CHUNK, BIGRAM_DIM), jnp.float32),
        pltpu.SemaphoreType.DMA,
    ],
)
def _sc_hash_gather(ids_hbm, table_hbm, out_hbm, ids_ext, hv, pairs_v, rows_v, sem_g):
    wid = lax.axis_index("s") * NC + lax.axis_index("c")
    base = wid * CHUNK

    pltpu.sync_copy(ids_hbm.at[pl.ds(base, CHUNK)], ids_ext.at[pl.ds(L, CHUNK)])

    @pl.when(wid != 0)
    def _():
        pltpu.sync_copy(ids_hbm.at[pl.ds(base - L, L)], ids_ext.at[pl.ds(0, L)])

    lane = lax.iota(jnp.int32, L)
    rs = 1 - jnp.clip(base % S, 0, 1)
    keep0 = 1 - rs * jnp.clip(1 - lane, 0, 1)
    for i in range(NVEC):
        cur = ids_ext[pl.ds(L + i * L, L)]
        prv = ids_ext[pl.ds(L - 1 + i * L, L)]
        if i == 0:
            prv = prv * keep0
        h = (prv * 1009 + cur) % N_BUCKETS
        hv[pl.ds(i * L, L)] = h

    def enqueue(t, _):
        h = hv[pl.ds(t, L)][0]
        prow = ((h >> 13) << 12) | (h & (_THALF - 1))
        pltpu.make_async_copy(
            table_hbm.at[pl.ds(prow, 1), :],
            pairs_v.at[pl.ds(t % GBATCH, 1), :],
            sem_g,
        ).start()
        return 0

    def drain(t, _):
        pltpu.make_async_copy(
            table_hbm.at[pl.ds(0, 1), :],
            pairs_v.at[pl.ds(0, 1), :],
            sem_g,
        ).wait()
        return 0

    def select(t, _):
        h = hv[pl.ds(t, L)][0]
        off = ((h >> 12) & 1) * BIGRAM_DIM
        for j in range(BIGRAM_DIM // L):
            rows_v[t, pl.ds(j * L, L)] = pairs_v[t % GBATCH, pl.ds(off + j * L, L)]
        return 0

    for b in range(NBATCH):
        lax.fori_loop(b * GBATCH, (b + 1) * GBATCH, enqueue, 0, unroll=8)
        lax.fori_loop(0, GBATCH, drain, 0, unroll=8)
        lax.fori_loop(b * GBATCH, (b + 1) * GBATCH, select, 0, unroll=4)

    pltpu.sync_copy(rows_v, out_hbm.at[pl.ds(base, CHUNK)])


def _tc_matmul_body(emb_ref, proj_ref, out_ref):
    out_ref[...] = lax.dot_general(
        emb_ref[...],
        proj_ref[...],
        (((1,), (1,)), ((), ())),
        preferred_element_type=jnp.float32,
    )


_ROWS_BLK = 2048
_tc_matmul = pl.pallas_call(
    _tc_matmul_body,
    grid=(N // _ROWS_BLK,),
    in_specs=[
        pl.BlockSpec((_ROWS_BLK, BIGRAM_DIM), lambda i: (i, 0)),
        pl.BlockSpec((D_MODEL, BIGRAM_DIM), lambda i: (0, 0)),
    ],
    out_specs=pl.BlockSpec((_ROWS_BLK, D_MODEL), lambda i: (i, 0)),
    out_shape=jax.ShapeDtypeStruct((N, D_MODEL), jnp.float32),
)


@jax.jit
def kernel(ids, embed_weight, proj_weight):
    ids_flat = ids.reshape(N).astype(jnp.int32)
    table_pairs = _tc_pack(embed_weight.T)
    emb = _sc_hash_gather(ids_flat, table_pairs)
    out = _tc_matmul(emb, proj_weight)
    return out.reshape(B, S, D_MODEL)


# speedup vs baseline: 1.5712x; 1.1045x over previous
import functools

import jax
import jax.numpy as jnp
from jax import lax
from jax.experimental import pallas as pl
from jax.experimental.pallas import tpu as pltpu
from jax.experimental.pallas import tpu_sc as plsc

N_BUCKETS = 1000000
BIGRAM_DIM = 64
D_MODEL = 1024
B, S = 4, 4096
N = B * S

NC, NS, L = 2, 16, 16
NW = NC * NS
CHUNK = N // NW
NVEC = CHUNK // L
GBATCH = 128
NBATCH = CHUNK // GBATCH
PAIRW = 2 * BIGRAM_DIM

_sc_mesh = plsc.VectorSubcoreMesh(core_axis_name="c", subcore_axis_name="s")


def _tc_pack_body(tab_t_ref, out_ref):
    out_ref[:, 0:BIGRAM_DIM] = tab_t_ref[:, 0:_THALF].T
    out_ref[:, BIGRAM_DIM:PAIRW] = tab_t_ref[:, _THALF:_TBLK].T


_TBLK = 16384
_THALF = _TBLK // 2
_TGRID = pl.cdiv(N_BUCKETS, _TBLK)
_PROWS = _TGRID * _THALF
_tc_pack = pl.pallas_call(
    _tc_pack_body,
    grid=(_TGRID,),
    in_specs=[pl.BlockSpec((BIGRAM_DIM, _TBLK), lambda i: (0, i))],
    out_specs=pl.BlockSpec((_THALF, PAIRW), lambda i: (i, 0)),
    out_shape=jax.ShapeDtypeStruct((_PROWS, PAIRW), jnp.float32),
)


@functools.partial(
    pl.kernel,
    out_type=jax.ShapeDtypeStruct((N, BIGRAM_DIM), jnp.float32),
    mesh=_sc_mesh,
    scratch_types=[
        pltpu.VMEM((CHUNK + L,), jnp.int32),
        pltpu.VMEM((CHUNK + L,), jnp.int32),
        pltpu.VMEM((GBATCH, PAIRW), jnp.float32),
        pltpu.VMEM((CHUNK, BIGRAM_DIM), jnp.float32),
        pltpu.SemaphoreType.DMA,
    ],
)
def _sc_hash_gather(ids_hbm, table_hbm, out_hbm, ids_ext, hv, pairs_v, rows_v, sem_g):
    wid = lax.axis_index("s") * NC + lax.axis_index("c")
    base = wid * CHUNK

    pltpu.sync_copy(ids_hbm.at[pl.ds(base, CHUNK)], ids_ext.at[pl.ds(L, CHUNK)])

    @pl.when(wid != 0)
    def _():
        pltpu.sync_copy(ids_hbm.at[pl.ds(base - L, L)], ids_ext.at[pl.ds(0, L)])

    lane = lax.iota(jnp.int32, L)
    rs = 1 - jnp.clip(base % S, 0, 1)
    keep0 = 1 - rs * jnp.clip(1 - lane, 0, 1)
    for i in range(NVEC):
        cur = ids_ext[pl.ds(L + i * L, L)]
        prv = ids_ext[pl.ds(L - 1 + i * L, L)]
        if i == 0:
            prv = prv * keep0
        h = (prv * 1009 + cur) % N_BUCKETS
        hv[pl.ds(i * L, L)] = h

    def enqueue(t, _):
        h = hv[pl.ds(t, L)][0]
        prow = ((h >> 14) << 13) | (h & (_THALF - 1))
        pltpu.make_async_copy(
            table_hbm.at[pl.ds(prow, 1), :],
            pairs_v.at[pl.ds(t % GBATCH, 1), :],
            sem_g,
        ).start()
        return 0

    def drain(t, _):
        pltpu.make_async_copy(
            table_hbm.at[pl.ds(0, 1), :],
            pairs_v.at[pl.ds(0, 1), :],
            sem_g,
        ).wait()
        return 0

    def select(t, _):
        h = hv[pl.ds(t, L)][0]
        off = ((h >> 13) & 1) * BIGRAM_DIM
        for j in range(BIGRAM_DIM // L):
            rows_v[t, pl.ds(j * L, L)] = pairs_v[t % GBATCH, pl.ds(off + j * L, L)]
        return 0

    for b in range(NBATCH):
        lax.fori_loop(b * GBATCH, (b + 1) * GBATCH, enqueue, 0, unroll=8)
        lax.fori_loop(0, GBATCH, drain, 0, unroll=8)
        lax.fori_loop(b * GBATCH, (b + 1) * GBATCH, select, 0, unroll=4)

    pltpu.sync_copy(rows_v, out_hbm.at[pl.ds(base, CHUNK)])


def _tc_matmul_body(emb_ref, proj_ref, out_ref):
    out_ref[...] = lax.dot_general(
        emb_ref[...],
        proj_ref[...],
        (((1,), (1,)), ((), ())),
        preferred_element_type=jnp.float32,
    )


_ROWS_BLK = 2048
_tc_matmul = pl.pallas_call(
    _tc_matmul_body,
    grid=(N // _ROWS_BLK,),
    in_specs=[
        pl.BlockSpec((_ROWS_BLK, BIGRAM_DIM), lambda i: (i, 0)),
        pl.BlockSpec((D_MODEL, BIGRAM_DIM), lambda i: (0, 0)),
    ],
    out_specs=pl.BlockSpec((_ROWS_BLK, D_MODEL), lambda i: (i, 0)),
    out_shape=jax.ShapeDtypeStruct((N, D_MODEL), jnp.float32),
)


@jax.jit
def kernel(ids, embed_weight, proj_weight):
    ids_flat = ids.reshape(N).astype(jnp.int32)
    table_pairs = _tc_pack(embed_weight.T)
    emb = _sc_hash_gather(ids_flat, table_pairs)
    out = _tc_matmul(emb, proj_weight)
    return out.reshape(B, S, D_MODEL)


# speedup vs baseline: 1.6564x; 1.0542x over previous
import functools

import jax
import jax.numpy as jnp
from jax import lax
from jax.experimental import pallas as pl
from jax.experimental.pallas import tpu as pltpu
from jax.experimental.pallas import tpu_sc as plsc

N_BUCKETS = 1000000
BIGRAM_DIM = 64
D_MODEL = 1024
B, S = 4, 4096
N = B * S

NC, NS, L = 2, 16, 16
NW = NC * NS
CHUNK = N // NW
NVEC = CHUNK // L
GBATCH = 128
NBATCH = CHUNK // GBATCH
PAIRW = 2 * BIGRAM_DIM

_sc_mesh = plsc.VectorSubcoreMesh(core_axis_name="c", subcore_axis_name="s")


def _tc_pack_body(tab_t_ref, out_ref):
    out_ref[:, 0:BIGRAM_DIM] = tab_t_ref[:, 0:_THALF].T
    out_ref[:, BIGRAM_DIM:PAIRW] = tab_t_ref[:, _THALF:_TBLK].T


_TBLK = 32768
_THALF = _TBLK // 2
_TGRID = pl.cdiv(N_BUCKETS, _TBLK)
_PROWS = _TGRID * _THALF
_tc_pack = pl.pallas_call(
    _tc_pack_body,
    grid=(_TGRID,),
    in_specs=[pl.BlockSpec((BIGRAM_DIM, _TBLK), lambda i: (0, i))],
    out_specs=pl.BlockSpec((_THALF, PAIRW), lambda i: (i, 0)),
    out_shape=jax.ShapeDtypeStruct((_PROWS, PAIRW), jnp.float32),
)


@functools.partial(
    pl.kernel,
    out_type=jax.ShapeDtypeStruct((N, BIGRAM_DIM), jnp.float32),
    mesh=_sc_mesh,
    scratch_types=[
        pltpu.VMEM((CHUNK + L,), jnp.int32),
        pltpu.VMEM((CHUNK + L,), jnp.int32),
        pltpu.VMEM((GBATCH, PAIRW), jnp.float32),
        pltpu.VMEM((CHUNK, BIGRAM_DIM), jnp.float32),
        pltpu.SemaphoreType.DMA,
    ],
)
def _sc_hash_gather(ids_hbm, table_hbm, out_hbm, ids_ext, hv, pairs_v, rows_v, sem_g):
    wid = lax.axis_index("s") * NC + lax.axis_index("c")
    base = wid * CHUNK

    pltpu.sync_copy(ids_hbm.at[pl.ds(base, CHUNK)], ids_ext.at[pl.ds(L, CHUNK)])

    @pl.when(wid != 0)
    def _():
        pltpu.sync_copy(ids_hbm.at[pl.ds(base - L, L)], ids_ext.at[pl.ds(0, L)])

    lane = lax.iota(jnp.int32, L)
    rs = 1 - jnp.clip(base % S, 0, 1)
    keep0 = 1 - rs * jnp.clip(1 - lane, 0, 1)
    for i in range(NVEC):
        cur = ids_ext[pl.ds(L + i * L, L)]
        prv = ids_ext[pl.ds(L - 1 + i * L, L)]
        if i == 0:
            prv = prv * keep0
        h = (prv * 1009 + cur) % N_BUCKETS
        hv[pl.ds(i * L, L)] = h

    def enqueue(t, _):
        h = hv[pl.ds(t, L)][0]
        prow = ((h >> 15) << 14) | (h & (_THALF - 1))
        pltpu.make_async_copy(
            table_hbm.at[pl.ds(prow, 1), :],
            pairs_v.at[pl.ds(t % GBATCH, 1), :],
            sem_g,
        ).start()
        return 0

    def drain(t, _):
        pltpu.make_async_copy(
            table_hbm.at[pl.ds(0, 1), :],
            pairs_v.at[pl.ds(0, 1), :],
            sem_g,
        ).wait()
        return 0

    def select(t, _):
        h = hv[pl.ds(t, L)][0]
        off = ((h >> 14) & 1) * BIGRAM_DIM
        for j in range(BIGRAM_DIM // L):
            rows_v[t, pl.ds(j * L, L)] = pairs_v[t % GBATCH, pl.ds(off + j * L, L)]
        return 0

    for b in range(NBATCH):
        lax.fori_loop(b * GBATCH, (b + 1) * GBATCH, enqueue, 0, unroll=8)
        lax.fori_loop(0, GBATCH, drain, 0, unroll=8)
        lax.fori_loop(b * GBATCH, (b + 1) * GBATCH, select, 0, unroll=4)

    pltpu.sync_copy(rows_v, out_hbm.at[pl.ds(base, CHUNK)])


def _tc_matmul_body(emb_ref, proj_ref, out_ref):
    out_ref[...] = lax.dot_general(
        emb_ref[...],
        proj_ref[...],
        (((1,), (1,)), ((), ())),
        preferred_element_type=jnp.float32,
    )


_ROWS_BLK = 2048
_tc_matmul = pl.pallas_call(
    _tc_matmul_body,
    grid=(N // _ROWS_BLK,),
    in_specs=[
        pl.BlockSpec((_ROWS_BLK, BIGRAM_DIM), lambda i: (i, 0)),
        pl.BlockSpec((D_MODEL, BIGRAM_DIM), lambda i: (0, 0)),
    ],
    out_specs=pl.BlockSpec((_ROWS_BLK, D_MODEL), lambda i: (i, 0)),
    out_shape=jax.ShapeDtypeStruct((N, D_MODEL), jnp.float32),
)


@jax.jit
def kernel(ids, embed_weight, proj_weight):
    ids_flat = ids.reshape(N).astype(jnp.int32)
    table_pairs = _tc_pack(embed_weight.T)
    emb = _sc_hash_gather(ids_flat, table_pairs)
    out = _tc_matmul(emb, proj_weight)
    return out.reshape(B, S, D_MODEL)
